# Initial kernel scaffold; baseline (speedup 1.0000x reference)
import jax
import jax.numpy as jnp
from jax.experimental import pallas as pl


def kernel(x, edge_index, W1, W2, Wr, br, Wf, bf):
    raise NotImplementedError("write your pallas kernel here")


# speedup vs baseline: 4.5139x; 4.5139x over previous
import functools

import jax
import jax.numpy as jnp
from jax import lax
from jax.experimental import pallas as pl
from jax.experimental.pallas import tpu as pltpu
from jax.experimental.pallas import tpu_sc as plsc

N = 10000
E = 320000
D = 128
L = 3

NC = 2
NS = 16
CHUNK = 128
N_PAD = 10240
ROWS_PER_TILE = N_PAD // NS
CHUNKS_PER_TILE = 157
E_PAD = NS * CHUNKS_PER_TILE * CHUNK


def _sc_aggregate_body(h_hbm, src_hbm, dst_hbm, sums_hbm,
                       idxg_v, idxs_v, rows_v, acc_sh, sem):
  c = lax.axis_index("c")
  s = lax.axis_index("s")

  def fill_body(i, carry):
    for j in range(D // 16):
      rows_v[i, pl.ds(j * 16, 16)] = jnp.zeros((16,), jnp.float32)
    return carry

  lax.fori_loop(0, CHUNK, fill_body, 0)
  base = s * ROWS_PER_TILE
  for k in range(ROWS_PER_TILE // CHUNK):
    pltpu.sync_copy(rows_v, acc_sh.at[pl.ds(base + k * CHUNK, CHUNK)])
  plsc.subcore_barrier()

  def run_direction(g_hbm, s_hbm):
    def chunk_body(j, carry):
      eb = s * (CHUNKS_PER_TILE * CHUNK) + j * CHUNK
      pltpu.sync_copy(g_hbm.at[pl.ds(eb, CHUNK)], idxg_v)
      pltpu.sync_copy(s_hbm.at[pl.ds(eb, CHUNK)], idxs_v)
      pltpu.async_copy(h_hbm.at[idxg_v], rows_v, sem).wait()
      pltpu.sync_copy(rows_v, acc_sh.at[idxs_v], add=True)
      return carry

    lax.fori_loop(0, CHUNKS_PER_TILE, chunk_body, 0)

  @pl.when(c == 0)
  def _():
    run_direction(src_hbm, dst_hbm)

  @pl.when(c == 1)
  def _():
    run_direction(dst_hbm, src_hbm)

  plsc.subcore_barrier()

  @pl.when(c == 0)
  def _():
    pltpu.sync_copy(acc_sh.at[pl.ds(base, ROWS_PER_TILE)],
                    sums_hbm.at[0].at[pl.ds(base, ROWS_PER_TILE)])

  @pl.when(c == 1)
  def _():
    pltpu.sync_copy(acc_sh.at[pl.ds(base, ROWS_PER_TILE)],
                    sums_hbm.at[1].at[pl.ds(base, ROWS_PER_TILE)])


_sc_aggregate = pl.kernel(
    _sc_aggregate_body,
    out_type=jax.ShapeDtypeStruct((2, N_PAD, D), jnp.float32),
    mesh=plsc.VectorSubcoreMesh(core_axis_name="c", subcore_axis_name="s"),
    scratch_types=[
        pltpu.VMEM((CHUNK,), jnp.int32),
        pltpu.VMEM((CHUNK,), jnp.int32),
        pltpu.VMEM((CHUNK, D), jnp.float32),
        pltpu.VMEM_SHARED((N_PAD, D), jnp.float32),
        pltpu.SemaphoreType.DMA,
    ],
)


def _sc_counts_body(src_hbm, dst_hbm, cnts_hbm, idxs_v, ones_v, zeros_v,
                    cnt_sh):
  c = lax.axis_index("c")
  s = lax.axis_index("s")

  for j in range(CHUNK // 16):
    ones_v[pl.ds(j * 16, 16)] = jnp.ones((16,), jnp.float32)
    zeros_v[pl.ds(j * 16, 16)] = jnp.zeros((16,), jnp.float32)

  base = s * ROWS_PER_TILE
  for k in range(ROWS_PER_TILE // CHUNK):
    pltpu.sync_copy(zeros_v, cnt_sh.at[pl.ds(base + k * CHUNK, CHUNK)])
  plsc.subcore_barrier()

  def run_direction(s_hbm):
    def chunk_body(j, carry):
      eb = s * (CHUNKS_PER_TILE * CHUNK) + j * CHUNK
      pltpu.sync_copy(s_hbm.at[pl.ds(eb, CHUNK)], idxs_v)
      pltpu.sync_copy(ones_v, cnt_sh.at[idxs_v], add=True)
      return carry

    lax.fori_loop(0, CHUNKS_PER_TILE, chunk_body, 0)

  @pl.when(c == 0)
  def _():
    run_direction(dst_hbm)

  @pl.when(c == 1)
  def _():
    run_direction(src_hbm)

  plsc.subcore_barrier()

  @pl.when(c == 0)
  def _():
    pltpu.sync_copy(cnt_sh.at[pl.ds(base, ROWS_PER_TILE)],
                    cnts_hbm.at[pl.ds(base, ROWS_PER_TILE)])

  @pl.when(c == 1)
  def _():
    pltpu.sync_copy(cnt_sh.at[pl.ds(base, ROWS_PER_TILE)],
                    cnts_hbm.at[pl.ds(N_PAD + base, ROWS_PER_TILE)])


_sc_counts = pl.kernel(
    _sc_counts_body,
    out_type=jax.ShapeDtypeStruct((2 * N_PAD,), jnp.float32),
    mesh=plsc.VectorSubcoreMesh(core_axis_name="c", subcore_axis_name="s"),
    scratch_types=[
        pltpu.VMEM((CHUNK,), jnp.int32),
        pltpu.VMEM((CHUNK,), jnp.float32),
        pltpu.VMEM((CHUNK,), jnp.float32),
        pltpu.VMEM_SHARED((N_PAD,), jnp.float32),
    ],
)


def _tc_layer_body(h_ref, s1_ref, s2_ref, c1_ref, c2_ref, w1t_ref, w2t_ref,
                   wrt_ref, br_ref, o_ref, *, blk):
  r1 = 1.0 / jnp.maximum(c1_ref[...], 1.0)
  r2 = 1.0 / jnp.maximum(c2_ref[...], 1.0)
  m1 = s1_ref[...] * r1
  m2 = s2_ref[...] * r2
  acc = jnp.dot(h_ref[...], wrt_ref[...], preferred_element_type=jnp.float32)
  acc = acc + jnp.dot(m1, w1t_ref[...], preferred_element_type=jnp.float32)
  acc = acc + jnp.dot(m2, w2t_ref[...], preferred_element_type=jnp.float32)
  acc = acc + br_ref[...]
  acc = jnp.maximum(acc, 0.0)
  rows = lax.broadcasted_iota(jnp.int32, acc.shape, 0) + pl.program_id(0) * blk
  o_ref[...] = jnp.where(rows < N, acc, 0.0)


def _tc_layer(h, s1, s2, c1, c2, w1t, w2t, wrt, brl):
  blk = 1024
  row_spec = lambda w: pl.BlockSpec((blk, w), lambda i: (i, 0))
  full_spec = lambda a, b: pl.BlockSpec((a, b), lambda i: (0, 0))
  return pl.pallas_call(
      functools.partial(_tc_layer_body, blk=blk),
      grid=(N_PAD // blk,),
      in_specs=[
          row_spec(D), row_spec(D), row_spec(D), row_spec(1), row_spec(1),
          full_spec(D, D), full_spec(D, D), full_spec(D, D), full_spec(1, D),
      ],
      out_specs=row_spec(D),
      out_shape=jax.ShapeDtypeStruct((N_PAD, D), jnp.float32),
  )(h, s1, s2, c1, c2, w1t, w2t, wrt, brl)


def _tc_final_body(x_ref, h1_ref, h2_ref, h3_ref, wft_ref, bf_ref, o_ref):
  cat = jnp.concatenate(
      [x_ref[...], h1_ref[...], h2_ref[...], h3_ref[...]], axis=1)
  o_ref[...] = jnp.dot(
      cat, wft_ref[...], preferred_element_type=jnp.float32) + bf_ref[...]


def _tc_final(x, h1, h2, h3, wft, bf):
  blk = 1024
  row_spec = pl.BlockSpec((blk, D), lambda i: (i, 0))
  return pl.pallas_call(
      _tc_final_body,
      grid=(N_PAD // blk,),
      in_specs=[
          row_spec, row_spec, row_spec, row_spec,
          pl.BlockSpec((L * D + D, D), lambda i: (0, 0)),
          pl.BlockSpec((1, D), lambda i: (0, 0)),
      ],
      out_specs=row_spec,
      out_shape=jax.ShapeDtypeStruct((N_PAD, D), jnp.float32),
  )(x, h1, h2, h3, wft, bf)


def kernel(x, edge_index, W1, W2, Wr, br, Wf, bf):
  x_pad = jnp.zeros((N_PAD, D), jnp.float32).at[:N].set(x)
  pad = jnp.full((E_PAD - E,), N, jnp.int32)
  src_p = jnp.concatenate([edge_index[0], pad])
  dst_p = jnp.concatenate([edge_index[1], pad])
  w1t = jnp.transpose(W1, (0, 2, 1))
  w2t = jnp.transpose(W2, (0, 2, 1))
  wrt = jnp.transpose(Wr, (0, 2, 1))
  wft = jnp.transpose(Wf)
  h = x_pad
  hs = []
  cnts = _sc_counts(src_p, dst_p).reshape(2, N_PAD, 1)
  for l in range(L):
    sums = _sc_aggregate(h, src_p, dst_p)
    h = _tc_layer(h, sums[0], sums[1], cnts[0], cnts[1],
                  w1t[l], w2t[l], wrt[l], br[l].reshape(1, D))
    hs.append(h)
  out = _tc_final(x_pad, hs[0], hs[1], hs[2], wft, bf.reshape(1, D))
  return out[:N]


# speedup vs baseline: 5.6208x; 1.2452x over previous
import functools

import jax
import jax.numpy as jnp
from jax import lax
from jax.experimental import pallas as pl
from jax.experimental.pallas import tpu as pltpu
from jax.experimental.pallas import tpu_sc as plsc

N = 10000
E = 320000
D = 128
L = 3

NC = 2
NS = 16
CHUNK = 96
N_PAD = 10240
ROWS_PER_TILE = N_PAD // NS
CPT = 210
E_PAD = NS * CPT * CHUNK
IDX_BITS = 14
IDX_MASK = (1 << IDX_BITS) - 1


def _unpack(c, pk_v, idxg_v, idxs_v):
  for k in range(CHUNK // 16):
    p = pk_v[pl.ds(k * 16, 16)]
    lo = p & IDX_MASK
    hi = lax.shift_right_logical(p, IDX_BITS)
    idxg_v[pl.ds(k * 16, 16)] = jnp.where(c == 0, lo, hi)
    idxs_v[pl.ds(k * 16, 16)] = jnp.where(c == 0, hi, lo)


def _sc_aggregate_body(h_hbm, edges_hbm, sums_hbm,
                       pk0_v, pk1_v, idxg0_v, idxs0_v, idxg1_v, idxs1_v,
                       rows0_v, rows1_v, acc_sh,
                       semp0, semp1, semg0, semg1):
  c = lax.axis_index("c")
  s = lax.axis_index("s")

  def fill_body(i, carry):
    for j in range(D // 16):
      rows0_v[i, pl.ds(j * 16, 16)] = jnp.zeros((16,), jnp.float32)
    return carry

  lax.fori_loop(0, CHUNK, fill_body, 0)
  base = s * ROWS_PER_TILE
  nz = ROWS_PER_TILE // CHUNK
  for k in range(nz):
    pltpu.sync_copy(rows0_v, acc_sh.at[pl.ds(base + k * CHUNK, CHUNK)])
  rem = ROWS_PER_TILE - nz * CHUNK
  if rem:
    pltpu.sync_copy(rows0_v.at[pl.ds(0, rem)],
                    acc_sh.at[pl.ds(base + nz * CHUNK, rem)])
  plsc.subcore_barrier()

  ebase = s * (CPT * CHUNK)

  def load_pk(jj, pk_buf, sem):
    pltpu.async_copy(edges_hbm.at[pl.ds(ebase + jj * CHUNK, CHUNK)],
                     pk_buf, sem)

  def drain_pk(pk_buf, sem):
    pltpu.make_async_copy(edges_hbm.at[pl.ds(0, CHUNK)], pk_buf, sem).wait()

  def fire(idxg_v, rows_buf, sem):
    pltpu.async_copy(h_hbm.at[idxg_v], rows_buf, sem)

  def drain_rows(rows_buf, sem):
    pltpu.make_async_copy(
        sums_hbm.at[0].at[pl.ds(0, CHUNK)], rows_buf, sem).wait()

  load_pk(0, pk0_v, semp0)

  def body(g, carry):
    j0 = 2 * g
    load_pk(j0 + 1, pk1_v, semp1)
    drain_pk(pk0_v, semp0)
    _unpack(c, pk0_v, idxg0_v, idxs0_v)
    fire(idxg0_v, rows0_v, semg0)

    @pl.when(g < CPT // 2 - 1)
    def _():
      load_pk(j0 + 2, pk0_v, semp0)

    drain_pk(pk1_v, semp1)
    _unpack(c, pk1_v, idxg1_v, idxs1_v)
    fire(idxg1_v, rows1_v, semg1)

    drain_rows(rows0_v, semg0)
    pltpu.sync_copy(rows0_v, acc_sh.at[idxs0_v], add=True)
    drain_rows(rows1_v, semg1)
    pltpu.sync_copy(rows1_v, acc_sh.at[idxs1_v], add=True)
    return carry

  lax.fori_loop(0, CPT // 2, body, 0)
  plsc.subcore_barrier()

  @pl.when(c == 0)
  def _():
    pltpu.sync_copy(acc_sh.at[pl.ds(base, ROWS_PER_TILE)],
                    sums_hbm.at[0].at[pl.ds(base, ROWS_PER_TILE)])

  @pl.when(c == 1)
  def _():
    pltpu.sync_copy(acc_sh.at[pl.ds(base, ROWS_PER_TILE)],
                    sums_hbm.at[1].at[pl.ds(base, ROWS_PER_TILE)])


_sc_aggregate = pl.kernel(
    _sc_aggregate_body,
    out_type=jax.ShapeDtypeStruct((2, N_PAD, D), jnp.float32),
    mesh=plsc.VectorSubcoreMesh(core_axis_name="c", subcore_axis_name="s"),
    scratch_types=[
        pltpu.VMEM((CHUNK,), jnp.int32),
        pltpu.VMEM((CHUNK,), jnp.int32),
        pltpu.VMEM((CHUNK,), jnp.int32),
        pltpu.VMEM((CHUNK,), jnp.int32),
        pltpu.VMEM((CHUNK,), jnp.int32),
        pltpu.VMEM((CHUNK,), jnp.int32),
        pltpu.VMEM((CHUNK, D), jnp.float32),
        pltpu.VMEM((CHUNK, D), jnp.float32),
        pltpu.VMEM_SHARED((N_PAD, D), jnp.float32),
        pltpu.SemaphoreType.DMA,
        pltpu.SemaphoreType.DMA,
        pltpu.SemaphoreType.DMA,
        pltpu.SemaphoreType.DMA,
    ],
)


def _sc_counts_body(edges_hbm, cnts_hbm, pk0_v, pk1_v, idxg_v, idxs0_v,
                    idxs1_v, ones_v, zeros_v, cnt_sh, semp0, semp1):
  c = lax.axis_index("c")
  s = lax.axis_index("s")

  for j in range(128 // 16):
    zeros_v[pl.ds(j * 16, 16)] = jnp.zeros((16,), jnp.float32)
  for j in range(CHUNK // 16):
    ones_v[pl.ds(j * 16, 16)] = jnp.ones((16,), jnp.float32)

  base = s * ROWS_PER_TILE
  for k in range(ROWS_PER_TILE // 128):
    pltpu.sync_copy(zeros_v, cnt_sh.at[pl.ds(base + k * 128, 128)])
  plsc.subcore_barrier()

  ebase = s * (CPT * CHUNK)

  def load_pk(jj, pk_buf, sem):
    pltpu.async_copy(edges_hbm.at[pl.ds(ebase + jj * CHUNK, CHUNK)],
                     pk_buf, sem)

  def drain_pk(pk_buf, sem):
    pltpu.make_async_copy(edges_hbm.at[pl.ds(0, CHUNK)], pk_buf, sem).wait()

  load_pk(0, pk0_v, semp0)

  def body(g, carry):
    j0 = 2 * g
    load_pk(j0 + 1, pk1_v, semp1)
    drain_pk(pk0_v, semp0)
    _unpack(c, pk0_v, idxg_v, idxs0_v)

    @pl.when(g < CPT // 2 - 1)
    def _():
      load_pk(j0 + 2, pk0_v, semp0)

    pltpu.sync_copy(ones_v, cnt_sh.at[idxs0_v], add=True)
    drain_pk(pk1_v, semp1)
    _unpack(c, pk1_v, idxg_v, idxs1_v)
    pltpu.sync_copy(ones_v, cnt_sh.at[idxs1_v], add=True)
    return carry

  lax.fori_loop(0, CPT // 2, body, 0)
  plsc.subcore_barrier()

  @pl.when(c == 0)
  def _():
    pltpu.sync_copy(cnt_sh.at[pl.ds(base, ROWS_PER_TILE)],
                    cnts_hbm.at[pl.ds(base, ROWS_PER_TILE)])

  @pl.when(c == 1)
  def _():
    pltpu.sync_copy(cnt_sh.at[pl.ds(base, ROWS_PER_TILE)],
                    cnts_hbm.at[pl.ds(N_PAD + base, ROWS_PER_TILE)])


_sc_counts = pl.kernel(
    _sc_counts_body,
    out_type=jax.ShapeDtypeStruct((2 * N_PAD,), jnp.float32),
    mesh=plsc.VectorSubcoreMesh(core_axis_name="c", subcore_axis_name="s"),
    scratch_types=[
        pltpu.VMEM((CHUNK,), jnp.int32),
        pltpu.VMEM((CHUNK,), jnp.int32),
        pltpu.VMEM((CHUNK,), jnp.int32),
        pltpu.VMEM((CHUNK,), jnp.int32),
        pltpu.VMEM((CHUNK,), jnp.int32),
        pltpu.VMEM((CHUNK,), jnp.float32),
        pltpu.VMEM((128,), jnp.float32),
        pltpu.VMEM_SHARED((N_PAD,), jnp.float32),
        pltpu.SemaphoreType.DMA,
        pltpu.SemaphoreType.DMA,
    ],
)


def _tc_layer_body(h_ref, s1_ref, s2_ref, c1_ref, c2_ref, w1t_ref, w2t_ref,
                   wrt_ref, br_ref, o_ref, *, blk):
  r1 = 1.0 / jnp.maximum(c1_ref[...], 1.0)
  r2 = 1.0 / jnp.maximum(c2_ref[...], 1.0)
  m1 = s1_ref[...] * r1
  m2 = s2_ref[...] * r2
  acc = jnp.dot(h_ref[...], wrt_ref[...], preferred_element_type=jnp.float32)
  acc = acc + jnp.dot(m1, w1t_ref[...], preferred_element_type=jnp.float32)
  acc = acc + jnp.dot(m2, w2t_ref[...], preferred_element_type=jnp.float32)
  acc = acc + br_ref[...]
  acc = jnp.maximum(acc, 0.0)
  rows = lax.broadcasted_iota(jnp.int32, acc.shape, 0) + pl.program_id(0) * blk
  o_ref[...] = jnp.where(rows < N, acc, 0.0)


def _tc_layer(h, s1, s2, c1, c2, w1t, w2t, wrt, brl):
  blk = 1024
  row_spec = lambda w: pl.BlockSpec((blk, w), lambda i: (i, 0))
  full_spec = lambda a, b: pl.BlockSpec((a, b), lambda i: (0, 0))
  return pl.pallas_call(
      functools.partial(_tc_layer_body, blk=blk),
      grid=(N_PAD // blk,),
      in_specs=[
          row_spec(D), row_spec(D), row_spec(D), row_spec(1), row_spec(1),
          full_spec(D, D), full_spec(D, D), full_spec(D, D), full_spec(1, D),
      ],
      out_specs=row_spec(D),
      out_shape=jax.ShapeDtypeStruct((N_PAD, D), jnp.float32),
  )(h, s1, s2, c1, c2, w1t, w2t, wrt, brl)


def _tc_final_body(x_ref, h1_ref, h2_ref, h3_ref, wft_ref, bf_ref, o_ref):
  cat = jnp.concatenate(
      [x_ref[...], h1_ref[...], h2_ref[...], h3_ref[...]], axis=1)
  o_ref[...] = jnp.dot(
      cat, wft_ref[...], preferred_element_type=jnp.float32) + bf_ref[...]


def _tc_final(x, h1, h2, h3, wft, bf):
  blk = 1024
  row_spec = pl.BlockSpec((blk, D), lambda i: (i, 0))
  return pl.pallas_call(
      _tc_final_body,
      grid=(N_PAD // blk,),
      in_specs=[
          row_spec, row_spec, row_spec, row_spec,
          pl.BlockSpec((L * D + D, D), lambda i: (0, 0)),
          pl.BlockSpec((1, D), lambda i: (0, 0)),
      ],
      out_specs=row_spec,
      out_shape=jax.ShapeDtypeStruct((N_PAD, D), jnp.float32),
  )(x, h1, h2, h3, wft, bf)


def kernel(x, edge_index, W1, W2, Wr, br, Wf, bf):
  x_pad = jnp.zeros((N_PAD, D), jnp.float32).at[:N].set(x)
  pad = jnp.full((E_PAD - E,), N, jnp.int32)
  src_p = jnp.concatenate([edge_index[0], pad])
  dst_p = jnp.concatenate([edge_index[1], pad])
  edges_p = src_p | (dst_p << IDX_BITS)
  w1t = jnp.transpose(W1, (0, 2, 1))
  w2t = jnp.transpose(W2, (0, 2, 1))
  wrt = jnp.transpose(Wr, (0, 2, 1))
  wft = jnp.transpose(Wf)
  h = x_pad
  hs = []
  cnts = _sc_counts(edges_p).reshape(2, N_PAD, 1)
  for l in range(L):
    sums = _sc_aggregate(h, edges_p)
    h = _tc_layer(h, sums[0], sums[1], cnts[0], cnts[1],
                  w1t[l], w2t[l], wrt[l], br[l].reshape(1, D))
    hs.append(h)
  out = _tc_final(x_pad, hs[0], hs[1], hs[2], wft, bf.reshape(1, D))
  return out[:N]
